# Initial kernel scaffold; baseline (speedup 1.0000x reference)
import jax
import jax.numpy as jnp
from jax.experimental import pallas as pl


def kernel(x, qkv_w, qkv_b, lepe_w, lepe_b, out_w, out_b):
    raise NotImplementedError("write your pallas kernel here")


# speedup vs baseline: 5.9222x; 5.9222x over previous
import functools

import jax
import jax.numpy as jnp
from jax.experimental import pallas as pl
from jax.experimental.pallas import tpu as pltpu

DIM = 192
NUM_HEADS = 8
HEAD_DIM = DIM // NUM_HEADS
N_WIN = 28
NREG = N_WIN * N_WIN
RH = 8
RS = RH * RH
TOPK = 4
SCALE = DIM ** (-0.5)
H = 224
W = 224

_RB = 16
_TR = 16


def _qkv_body(x_ref, w_ref, b_ref, q_ref, k_ref, v_ref, qr_ref, kr_ref):
    xm = x_ref[...].reshape(_RB * RS, DIM)
    y = jnp.dot(xm, w_ref[...], preferred_element_type=jnp.float32) + b_ref[...]
    q = y[:, :DIM].reshape(_RB, RS, DIM)
    k = y[:, DIM:2 * DIM].reshape(_RB, RS, DIM)
    v = y[:, 2 * DIM:].reshape(_RB, RS, DIM)
    q_ref[...] = q
    k_ref[...] = k
    v_ref[...] = v
    qr_ref[...] = jnp.mean(q, axis=1)
    kr_ref[...] = jnp.mean(k, axis=1)


def _qkv_call(xt, wq, b2):
    nsteps = NREG // _RB
    return pl.pallas_call(
        _qkv_body,
        grid=(nsteps,),
        in_specs=[
            pl.BlockSpec((_RB, RS, DIM), lambda i: (i, 0, 0)),
            pl.BlockSpec((DIM, 3 * DIM), lambda i: (0, 0)),
            pl.BlockSpec((1, 3 * DIM), lambda i: (0, 0)),
        ],
        out_specs=[
            pl.BlockSpec((_RB, RS, DIM), lambda i: (i, 0, 0)),
            pl.BlockSpec((_RB, RS, DIM), lambda i: (i, 0, 0)),
            pl.BlockSpec((_RB, RS, DIM), lambda i: (i, 0, 0)),
            pl.BlockSpec((_RB, DIM), lambda i: (i, 0)),
            pl.BlockSpec((_RB, DIM), lambda i: (i, 0)),
        ],
        out_shape=[
            jax.ShapeDtypeStruct((NREG, RS, DIM), jnp.float32),
            jax.ShapeDtypeStruct((NREG, RS, DIM), jnp.float32),
            jax.ShapeDtypeStruct((NREG, RS, DIM), jnp.float32),
            jax.ShapeDtypeStruct((NREG, DIM), jnp.float32),
            jax.ShapeDtypeStruct((NREG, DIM), jnp.float32),
        ],
    )(xt, wq, b2)


def _route_body(qr_ref, kr_ref, idx_ref):
    a = jax.lax.dot_general(
        qr_ref[...], kr_ref[...], (((1,), (1,)), ((), ())),
        preferred_element_type=jnp.float32)
    lane = jax.lax.broadcasted_iota(jnp.int32, (NREG, NREG), 1)
    for t in range(TOPK):
        mx = jnp.max(a, axis=1, keepdims=True)
        m = jnp.min(jnp.where(a == mx, lane, NREG), axis=1)
        idx_ref[t, :] = m
        a = jnp.where(lane == m[:, None], -jnp.inf, a)


def _route_call(qr, kr):
    return pl.pallas_call(
        _route_body,
        in_specs=[
            pl.BlockSpec((NREG, DIM), lambda: (0, 0)),
            pl.BlockSpec((NREG, DIM), lambda: (0, 0)),
        ],
        out_specs=pl.BlockSpec((TOPK, NREG), lambda: (0, 0)),
        out_shape=jax.ShapeDtypeStruct((TOPK, NREG), jnp.int32),
    )(qr, kr)


def _attn_body(idx_ref, q_ref, k0, k1, k2, k3, v0, v1, v2, v3, o_ref):
    del idx_ref
    q = q_ref[0] * SCALE
    kc = jnp.concatenate([k0[0], k1[0], k2[0], k3[0]], axis=0)
    vc = jnp.concatenate([v0[0], v1[0], v2[0], v3[0]], axis=0)
    outs = []
    for h in range(NUM_HEADS):
        s = h * HEAD_DIM
        qh = q[:, s:s + HEAD_DIM]
        kh = kc[:, s:s + HEAD_DIM]
        logits = jax.lax.dot_general(
            qh, kh, (((1,), (1,)), ((), ())),
            preferred_element_type=jnp.float32)
        mx = jnp.max(logits, axis=1, keepdims=True)
        p = jnp.exp(logits - mx)
        p = p / jnp.sum(p, axis=1, keepdims=True)
        outs.append(jnp.dot(p, vc[:, s:s + HEAD_DIM],
                            preferred_element_type=jnp.float32))
    o_ref[0] = jnp.concatenate(outs, axis=1)


def _gspec(t):
    return pl.BlockSpec((1, RS, DIM), lambda r, idx: (idx[t, r], 0, 0))


def _attn_call(idxs, q, k, v):
    grid_spec = pltpu.PrefetchScalarGridSpec(
        num_scalar_prefetch=1,
        grid=(NREG,),
        in_specs=[
            pl.BlockSpec((1, RS, DIM), lambda r, idx: (r, 0, 0)),
            _gspec(0), _gspec(1), _gspec(2), _gspec(3),
            _gspec(0), _gspec(1), _gspec(2), _gspec(3),
        ],
        out_specs=pl.BlockSpec((1, RS, DIM), lambda r, idx: (r, 0, 0)),
    )
    return pl.pallas_call(
        _attn_body,
        grid_spec=grid_spec,
        out_shape=jax.ShapeDtypeStruct((NREG, RS, DIM), jnp.float32),
    )(idxs, q, k, k, k, k, v, v, v, v)


def _out_body(vc_ref, vt_ref, vb_ref, a_ref, wl_ref, lb_ref, wo_ref, ob_ref,
              o_ref):
    i = pl.program_id(0)
    n = pl.num_programs(0)
    top = jnp.where(i > 0, vt_ref[...], 0.0)
    bot = jnp.where(i < n - 1, vb_ref[...], 0.0)
    ext = jnp.concatenate([top, vc_ref[...], bot], axis=0)
    acc = a_ref[...] + lb_ref[...][None]
    kk = 0
    for dh in (0, 1, 2):
        sh_rows = ext[dh:dh + _TR]
        for dw in (-1, 0, 1):
            if dw == -1:
                sh = jnp.concatenate(
                    [jnp.zeros((_TR, 1, DIM), jnp.float32), sh_rows[:, :-1]],
                    axis=1)
            elif dw == 1:
                sh = jnp.concatenate(
                    [sh_rows[:, 1:], jnp.zeros((_TR, 1, DIM), jnp.float32)],
                    axis=1)
            else:
                sh = sh_rows
            acc = acc + sh * wl_ref[kk, :][None, None, :]
            kk += 1
    y = acc.reshape(_TR * W, DIM)
    out = jax.lax.dot_general(
        wo_ref[...], y, (((1,), (1,)), ((), ())),
        preferred_element_type=jnp.float32)
    o_ref[...] = out.reshape(DIM, _TR, W) + ob_ref[...][:, :, None]


def _out_call(v_sp, a_sp, wl, lb, wo, ob):
    nsteps = H // _TR
    return pl.pallas_call(
        _out_body,
        grid=(nsteps,),
        in_specs=[
            pl.BlockSpec((_TR, W, DIM), lambda i: (i, 0, 0)),
            pl.BlockSpec((1, W, DIM), lambda i: (jnp.maximum(i * _TR - 1, 0), 0, 0)),
            pl.BlockSpec((1, W, DIM), lambda i: (jnp.minimum(i * _TR + _TR, H - 1), 0, 0)),
            pl.BlockSpec((_TR, W, DIM), lambda i: (i, 0, 0)),
            pl.BlockSpec((16, DIM), lambda i: (0, 0)),
            pl.BlockSpec((1, DIM), lambda i: (0, 0)),
            pl.BlockSpec((DIM, DIM), lambda i: (0, 0)),
            pl.BlockSpec((DIM, 1), lambda i: (0, 0)),
        ],
        out_specs=pl.BlockSpec((DIM, _TR, W), lambda i: (0, i, 0)),
        out_shape=jax.ShapeDtypeStruct((DIM, H, W), jnp.float32),
    )(v_sp, v_sp, v_sp, a_sp, wl, lb, wo, ob)


def kernel(x, qkv_w, qkv_b, lepe_w, lepe_b, out_w, out_b):
    xt = x.reshape(DIM, N_WIN, RH, N_WIN, RH).transpose(1, 3, 2, 4, 0)
    xt = xt.reshape(NREG, RS, DIM)
    wq = qkv_w.T
    b2 = qkv_b.reshape(1, 3 * DIM)

    q, k, v, qr, kr = _qkv_call(xt, wq, b2)
    idxs = _route_call(qr, kr)
    ot = _attn_call(idxs, q, k, v)

    def to_sp(t):
        t = t.reshape(N_WIN, N_WIN, RH, RH, DIM).transpose(0, 2, 1, 3, 4)
        return t.reshape(H, W, DIM)

    v_sp = to_sp(v)
    a_sp = to_sp(ot)
    wl = jnp.pad(lepe_w.reshape(DIM, 9).T, ((0, 7), (0, 0)))
    out = _out_call(v_sp, a_sp, wl, lepe_b.reshape(1, DIM), out_w,
                    out_b.reshape(DIM, 1))
    return out.reshape(1, DIM, H, W)
